# baseline (speedup 1.0000x reference)
import jax
import jax.numpy as jnp
import numpy as np
from jax import lax
from jax.experimental import pallas as pl
from jax.experimental.pallas import tpu as pltpu
from jax.experimental.pallas import tpu_sc as plsc

N = 10000
E = 320000
D_IN = 128
D_HID = 256
N_CLS = 2

NC = 2
NS = 16
NW = NC * NS

TBL_W = D_HID + 32
C1 = 80
C2 = 80
E_PER_W = E // NW
E_PER_T = E // NS
ROWS_PER_T = N // NS


def _tc_prep_body(x_ref, w1_ref, b1_ref, tn_ref, lohi_ref):
    x = x_ref[...]
    h = lax.dot_general(x, w1_ref[...], (((1,), (1,)), ((), ())),
                        precision=lax.Precision.HIGHEST)
    h = jnp.maximum(h + b1_ref[...][None, :], 0.0)
    ss = jnp.sum(h * h, axis=1, keepdims=True)
    norm = jnp.sqrt(ss)
    hn = h / jnp.maximum(norm, 1e-12)
    tn_ref[:, 0:D_HID] = hn.astype(jnp.bfloat16)
    tn_ref[:, D_HID:TBL_W] = jnp.zeros(
        (hn.shape[0], 32), jnp.bfloat16)
    lohi_ref[0] = h[:, :128].astype(jnp.bfloat16)
    lohi_ref[1] = h[:, 128:].astype(jnp.bfloat16)


def _tc_prep(x, W1, b1):
    blk = 1000
    grid = N // blk
    return pl.pallas_call(
        _tc_prep_body,
        grid=(grid,),
        in_specs=[
            pl.BlockSpec((blk, D_IN), lambda i: (i, 0)),
            pl.BlockSpec((D_HID, D_IN), lambda i: (0, 0)),
            pl.BlockSpec((D_HID,), lambda i: (0,)),
        ],
        out_specs=[
            pl.BlockSpec((blk, TBL_W), lambda i: (i, 0)),
            pl.BlockSpec((2, blk, 128), lambda i: (0, i, 0)),
        ],
        out_shape=[
            jax.ShapeDtypeStruct((N, TBL_W), jnp.bfloat16),
            jax.ShapeDtypeStruct((2, N, 128), jnp.bfloat16),
        ],
    )(x, W1, b1)


NCH1 = E_PER_W // C1


def _sc_pass1_body(tn_hbm, src_hbm, dst_hbm, beta_hbm,
                   ee_hbm,
                   sidx_all, didx_all, ee_all,
                   srowsA, drowsA, srowsB, drowsB, psum_v, beta_v,
                   semA1, semA2, semB1, semB2):
    wid = lax.axis_index("s") * NC + lax.axis_index("c")
    ebase = wid * E_PER_W
    pltpu.sync_copy(beta_hbm, beta_v)
    beta = beta_v[...]
    iota = lax.iota(jnp.int32, 16)
    pltpu.sync_copy(src_hbm.at[pl.ds(ebase, E_PER_W)], sidx_all)
    pltpu.sync_copy(dst_hbm.at[pl.ds(ebase, E_PER_W)], didx_all)

    def start(c, srows, drows, s1, s2):
        pltpu.async_copy(tn_hbm.at[sidx_all.at[pl.ds(c * C1, C1)]], srows, s1)
        pltpu.async_copy(tn_hbm.at[didx_all.at[pl.ds(c * C1, C1)]], drows, s2)

    def wait(srows, drows, s1, s2):
        pltpu.make_async_copy(tn_hbm.at[pl.ds(0, C1)], srows, s1).wait()
        pltpu.make_async_copy(tn_hbm.at[pl.ds(0, C1)], drows, s2).wait()

    def compute(c, srows, drows):
        def edge_body(e, _):
            acc0 = jnp.zeros((16,), jnp.float32)
            acc1 = jnp.zeros((16,), jnp.float32)
            for k in range(8):
                sw = srows[e, pl.ds(k * 32, 32)]
                dw = drows[e, pl.ds(k * 32, 32)]
                sa, sb = plsc.unpack(sw, format=plsc.PackFormat.INTERLEAVED)
                da, db = plsc.unpack(dw, format=plsc.PackFormat.INTERLEAVED)
                acc0 = acc0 + sa * da
                acc1 = acc1 + sb * db
            psum_v[e, pl.ds(0, 16)] = acc0 + acc1
            return 0

        lax.fori_loop(0, C1, edge_body, 0, unroll=4)
        for g in range(C1 // 16):
            row16 = g * 16 + iota
            tot = jnp.zeros((16,), jnp.float32)
            for k in range(16):
                tot = tot + plsc.load_gather(
                    psum_v, [row16, jnp.full((16,), k, jnp.int32)])
            e16 = jnp.exp(beta * tot)
            ee_all[pl.ds(c * C1 + g * 16, 16)] = e16

    start(0, srowsA, drowsA, semA1, semA2)

    def pair_body(it, _):
        cA = 2 * it
        wait(srowsA, drowsA, semA1, semA2)

        @pl.when(it < NCH1 // 2)
        def _():
            start(cA + 1, srowsB, drowsB, semB1, semB2)

        compute(cA, srowsA, drowsA)

        @pl.when(it < NCH1 // 2)
        def _():
            wait(srowsB, drowsB, semB1, semB2)
            start(cA + 2, srowsA, drowsA, semA1, semA2)
            compute(cA + 1, srowsB, drowsB)

        return 0

    lax.fori_loop(0, NCH1 // 2 + 1, pair_body, 0)
    pltpu.sync_copy(ee_all, ee_hbm.at[pl.ds(ebase, E_PER_W)])


def _sc_pass1(tn, src, dst, beta16):
    mesh = plsc.VectorSubcoreMesh(core_axis_name="c", subcore_axis_name="s")
    f = pl.kernel(
        _sc_pass1_body,
        out_type=[
            jax.ShapeDtypeStruct((E,), jnp.float32),
        ],
        mesh=mesh,
        scratch_types=[
            pltpu.VMEM((E_PER_W,), jnp.int32),
            pltpu.VMEM((E_PER_W,), jnp.int32),
            pltpu.VMEM((E_PER_W,), jnp.float32),
            pltpu.VMEM((C1, TBL_W), jnp.bfloat16),
            pltpu.VMEM((C1, TBL_W), jnp.bfloat16),
            pltpu.VMEM((C1, TBL_W), jnp.bfloat16),
            pltpu.VMEM((C1, TBL_W), jnp.bfloat16),
            pltpu.VMEM((C1, 17), jnp.float32),
            pltpu.VMEM((16,), jnp.float32),
            pltpu.SemaphoreType.DMA,
            pltpu.SemaphoreType.DMA,
            pltpu.SemaphoreType.DMA,
            pltpu.SemaphoreType.DMA,
        ],
        compiler_params=pltpu.CompilerParams(use_tc_tiling_on_sc=False,
                                             needs_layout_passes=False),
    )
    return f(tn, src, dst, beta16)


S2 = 2000
NST = E_PER_T // S2
NCH2 = S2 // C2


def _sc_pass2_body(hn2_hbm, src_hbm, dst_hbm, ee_hbm,
                   acc_hbm, den_hbm,
                   sidx_st, didx_st, ee_st,
                   rowsA, rowsB, frows_v, didx_v, denrows_v, zrows_v,
                   acc_sp, den_sp, semA, semB):
    cid = lax.axis_index("c")
    tid = lax.axis_index("s")
    iota = lax.iota(jnp.int32, 16)
    zero16 = jnp.zeros((16,), jnp.float32)

    def zb(r, _):
        for k in range(8):
            zrows_v[r, pl.ds(k * 16, 16)] = zero16
        return 0
    lax.fori_loop(0, 25, zb, 0)

    def zd(r, _):
        denrows_v[r, pl.ds(0, 16)] = zero16
        return 0
    lax.fori_loop(0, C2, zd, 0)

    r0 = tid * ROWS_PER_T
    for j in range(25):
        pltpu.sync_copy(zrows_v, acc_sp.at[pl.ds(r0 + j * 25, 25)])

    @pl.when(cid == 0)
    def _():
        for j in range(25):
            pltpu.sync_copy(zrows_v.at[pl.ds(0, 25), pl.ds(0, 16)],
                            den_sp.at[pl.ds(r0 + j * 25, 25)])

    plsc.subcore_barrier()

    ebase = tid * E_PER_T
    coff = cid * N

    def start(c, rows, sem):
        pltpu.async_copy(hn2_hbm.at[sidx_st.at[pl.ds(c * C2, C2)]], rows, sem)

    def wait(rows, sem):
        pltpu.make_async_copy(hn2_hbm.at[pl.ds(0, C2)], rows, sem).wait()

    def compute(c, rows):
        o = c * C2
        for j in range(C2 // 16):
            didx_v[pl.ds(j * 16, 16)] = didx_st[pl.ds(o + j * 16, 16)]

        def scale_body(e, _):
            w = plsc.load_gather(ee_st, [jnp.full((16,), o + e, jnp.int32)])
            for k in range(4):
                bw = rows[e, pl.ds(k * 32, 32)]
                a, b = plsc.unpack(bw, format=plsc.PackFormat.INTERLEAVED)
                frows_v[e, pl.ds(k * 32, 16)] = a * w
                frows_v[e, pl.ds(k * 32 + 16, 16)] = b * w
            return 0
        lax.fori_loop(0, C2, scale_body, 0, unroll=4)

        pltpu.sync_copy(frows_v, acc_sp.at[didx_v], add=True)

        @pl.when(cid == 0)
        def _():
            for g in range(C2 // 16):
                e16 = ee_st[pl.ds(o + g * 16, 16)]
                plsc.store_scatter(
                    denrows_v, [g * 16 + iota, jnp.zeros((16,), jnp.int32)],
                    e16)
            pltpu.sync_copy(denrows_v, den_sp.at[didx_v], add=True)

    def stage_body(st, _):
        sb = ebase + st * S2
        pltpu.sync_copy(src_hbm.at[pl.ds(sb, S2)], sidx_st)
        pltpu.sync_copy(dst_hbm.at[pl.ds(sb, S2)], didx_st)
        pltpu.sync_copy(ee_hbm.at[pl.ds(sb, S2)], ee_st)

        def shift_body(j, _):
            v = sidx_st[pl.ds(j * 16, 16)]
            sidx_st[pl.ds(j * 16, 16)] = v + coff
            return 0
        lax.fori_loop(0, S2 // 16, shift_body, 0, unroll=4)

        start(0, rowsA, semA)

        def pair_body(it, _):
            cA = 2 * it
            wait(rowsA, semA)

            @pl.when(it < NCH2 // 2)
            def _():
                start(cA + 1, rowsB, semB)

            compute(cA, rowsA)

            @pl.when(it < NCH2 // 2)
            def _():
                wait(rowsB, semB)
                start(cA + 2, rowsA, semA)
                compute(cA + 1, rowsB)

            return 0

        lax.fori_loop(0, NCH2 // 2 + 1, pair_body, 0)
        return 0

    lax.fori_loop(0, NST, stage_body, 0)
    plsc.subcore_barrier()

    pltpu.sync_copy(acc_sp.at[pl.ds(r0, ROWS_PER_T)],
                    acc_hbm.at[cid, pl.ds(r0, ROWS_PER_T)])

    @pl.when(cid == 0)
    def _():
        pltpu.sync_copy(den_sp.at[pl.ds(r0, ROWS_PER_T)],
                        den_hbm.at[pl.ds(r0, ROWS_PER_T)])


def _sc_pass2(hn2, src, dst, ee):
    mesh = plsc.VectorSubcoreMesh(core_axis_name="c", subcore_axis_name="s")
    f = pl.kernel(
        _sc_pass2_body,
        out_type=[
            jax.ShapeDtypeStruct((NC, N, 128), jnp.float32),
            jax.ShapeDtypeStruct((N, 16), jnp.float32),
        ],
        mesh=mesh,
        scratch_types=[
            pltpu.VMEM((S2,), jnp.int32),
            pltpu.VMEM((S2,), jnp.int32),
            pltpu.VMEM((S2,), jnp.float32),
            pltpu.VMEM((C2, 128), jnp.bfloat16),
            pltpu.VMEM((C2, 128), jnp.bfloat16),
            pltpu.VMEM((C2, 128), jnp.float32),
            pltpu.VMEM((C2,), jnp.int32),
            pltpu.VMEM((C2, 16), jnp.float32),
            pltpu.VMEM((25, 128), jnp.float32),
            pltpu.VMEM_SHARED((N, 128), jnp.float32),
            pltpu.VMEM_SHARED((N, 16), jnp.float32),
            pltpu.SemaphoreType.DMA,
            pltpu.SemaphoreType.DMA,
        ],
        compiler_params=pltpu.CompilerParams(use_tc_tiling_on_sc=False,
                                             needs_layout_passes=False),
    )
    return f(hn2, src, dst, ee)


def _tc_final_body(alo_ref, ahi_ref, den_ref, w2_ref, b2_ref, out_ref):
    den = den_ref[:, 0:1]
    inv = 1.0 / jnp.maximum(den, 1e-16)
    olo = alo_ref[0] * inv
    ohi = ahi_ref[0] * inv
    w2 = w2_ref[...]
    logits = (lax.dot_general(olo, w2[:, :128], (((1,), (1,)), ((), ())),
                              precision=lax.Precision.HIGHEST)
              + lax.dot_general(ohi, w2[:, 128:], (((1,), (1,)), ((), ())),
                                precision=lax.Precision.HIGHEST))
    logits = logits + b2_ref[...][None, :]
    m = jnp.max(logits, axis=1, keepdims=True)
    lse = jnp.log(jnp.sum(jnp.exp(logits - m), axis=1, keepdims=True)) + m
    out_ref[...] = logits - lse


def _tc_final(acc2, den, W2, b2):
    blk = 1000
    grid = N // blk
    return pl.pallas_call(
        _tc_final_body,
        grid=(grid,),
        in_specs=[
            pl.BlockSpec((1, blk, 128), lambda i: (0, i, 0)),
            pl.BlockSpec((1, blk, 128), lambda i: (1, i, 0)),
            pl.BlockSpec((blk, 16), lambda i: (i, 0)),
            pl.BlockSpec((N_CLS, D_HID), lambda i: (0, 0)),
            pl.BlockSpec((N_CLS,), lambda i: (0,)),
        ],
        out_specs=pl.BlockSpec((blk, N_CLS), lambda i: (i, 0)),
        out_shape=jax.ShapeDtypeStruct((N, N_CLS), jnp.float32),
    )(acc2, acc2, den, W2, b2)


_PERM = np.concatenate(
    [np.concatenate([b * 32 + 2 * np.arange(16),
                     b * 32 + 2 * np.arange(16) + 1])
     for b in range(8)]).astype(np.int32)


def kernel(x, edge_index, edge_weight, W1, b1, beta, W2, b2):
    del edge_weight
    ei = jnp.asarray(edge_index, jnp.int32)
    src = ei[0]
    dst = ei[1]
    beta16 = jnp.broadcast_to(beta.astype(jnp.float32), (16,))

    tn, lohi = _tc_prep(x, W1, b1)
    hn2 = lohi.reshape(2 * N, 128)

    (ee,) = _sc_pass1(tn, src, dst, beta16)
    acc2, den = _sc_pass2(hn2, src, dst, ee)

    return _tc_final(acc2, den, W2[:, _PERM], b2)


# speedup vs baseline: 1.4046x; 1.4046x over previous
import jax
import jax.numpy as jnp
import numpy as np
from jax import lax
from jax.experimental import pallas as pl
from jax.experimental.pallas import tpu as pltpu
from jax.experimental.pallas import tpu_sc as plsc

N = 10000
E = 320000
D_IN = 128
D_HID = 256
N_CLS = 2

NC = 2
NS = 16
NW = NC * NS

TBL_W = D_HID + 32
C1 = 80
C2 = 80
E_PER_W = E // NW
E_PER_T = E // NS
ROWS_PER_T = N // NS


def _tc_prep_body(x_ref, w1_ref, b1_ref, tn_ref, lohi_ref):
    x = x_ref[...]
    h = lax.dot_general(x, w1_ref[...], (((1,), (1,)), ((), ())),
                        precision=lax.Precision.HIGHEST)
    h = jnp.maximum(h + b1_ref[...][None, :], 0.0)
    ss = jnp.sum(h * h, axis=1, keepdims=True)
    norm = jnp.sqrt(ss)
    hn = h / jnp.maximum(norm, 1e-12)
    tn_ref[:, 0:D_HID] = hn.astype(jnp.bfloat16)
    tn_ref[:, D_HID:TBL_W] = jnp.zeros(
        (hn.shape[0], 32), jnp.bfloat16)
    lohi_ref[0] = h[:, :128]
    lohi_ref[1] = h[:, 128:]


def _tc_prep(x, W1, b1):
    blk = 1000
    grid = N // blk
    return pl.pallas_call(
        _tc_prep_body,
        grid=(grid,),
        in_specs=[
            pl.BlockSpec((blk, D_IN), lambda i: (i, 0)),
            pl.BlockSpec((D_HID, D_IN), lambda i: (0, 0)),
            pl.BlockSpec((D_HID,), lambda i: (0,)),
        ],
        out_specs=[
            pl.BlockSpec((blk, TBL_W), lambda i: (i, 0)),
            pl.BlockSpec((2, blk, 128), lambda i: (0, i, 0)),
        ],
        out_shape=[
            jax.ShapeDtypeStruct((N, TBL_W), jnp.bfloat16),
            jax.ShapeDtypeStruct((2, N, 128), jnp.float32),
        ],
    )(x, W1, b1)


NCH1 = E_PER_W // C1


def _sc_pass1_body(tn_hbm, src_hbm, dst_hbm, beta_hbm,
                   ee_hbm,
                   sidx_all, didx_all, ee_all,
                   srowsA, drowsA, srowsB, drowsB, psum_v, beta_v,
                   semA1, semA2, semB1, semB2):
    wid = lax.axis_index("s") * NC + lax.axis_index("c")
    ebase = wid * E_PER_W
    pltpu.sync_copy(beta_hbm, beta_v)
    beta = beta_v[...]
    iota = lax.iota(jnp.int32, 16)
    pltpu.sync_copy(src_hbm.at[pl.ds(ebase, E_PER_W)], sidx_all)
    pltpu.sync_copy(dst_hbm.at[pl.ds(ebase, E_PER_W)], didx_all)

    def start(c, srows, drows, s1, s2):
        pltpu.async_copy(tn_hbm.at[sidx_all.at[pl.ds(c * C1, C1)]], srows, s1)
        pltpu.async_copy(tn_hbm.at[didx_all.at[pl.ds(c * C1, C1)]], drows, s2)

    def wait(srows, drows, s1, s2):
        pltpu.make_async_copy(tn_hbm.at[pl.ds(0, C1)], srows, s1).wait()
        pltpu.make_async_copy(tn_hbm.at[pl.ds(0, C1)], drows, s2).wait()

    def compute(c, srows, drows):
        def edge_body(e, _):
            acc0 = jnp.zeros((16,), jnp.float32)
            acc1 = jnp.zeros((16,), jnp.float32)
            for k in range(8):
                sw = srows[e, pl.ds(k * 32, 32)]
                dw = drows[e, pl.ds(k * 32, 32)]
                sa, sb = plsc.unpack(sw, format=plsc.PackFormat.INTERLEAVED)
                da, db = plsc.unpack(dw, format=plsc.PackFormat.INTERLEAVED)
                acc0 = acc0 + sa * da
                acc1 = acc1 + sb * db
            psum_v[e, pl.ds(0, 16)] = acc0 + acc1
            return 0

        lax.fori_loop(0, C1, edge_body, 0, unroll=4)
        for g in range(C1 // 16):
            row16 = g * 16 + iota
            tot = jnp.zeros((16,), jnp.float32)
            for k in range(16):
                tot = tot + plsc.load_gather(
                    psum_v, [row16, jnp.full((16,), k, jnp.int32)])
            e16 = jnp.exp(beta * tot)
            ee_all[pl.ds(c * C1 + g * 16, 16)] = e16

    start(0, srowsA, drowsA, semA1, semA2)

    def pair_body(it, _):
        cA = 2 * it
        wait(srowsA, drowsA, semA1, semA2)

        @pl.when(it < NCH1 // 2)
        def _():
            start(cA + 1, srowsB, drowsB, semB1, semB2)

        compute(cA, srowsA, drowsA)

        @pl.when(it < NCH1 // 2)
        def _():
            wait(srowsB, drowsB, semB1, semB2)
            start(cA + 2, srowsA, drowsA, semA1, semA2)
            compute(cA + 1, srowsB, drowsB)

        return 0

    lax.fori_loop(0, NCH1 // 2 + 1, pair_body, 0)
    pltpu.sync_copy(ee_all, ee_hbm.at[pl.ds(ebase, E_PER_W)])


def _sc_pass1(tn, src, dst, beta16):
    mesh = plsc.VectorSubcoreMesh(core_axis_name="c", subcore_axis_name="s")
    f = pl.kernel(
        _sc_pass1_body,
        out_type=[
            jax.ShapeDtypeStruct((E,), jnp.float32),
        ],
        mesh=mesh,
        scratch_types=[
            pltpu.VMEM((E_PER_W,), jnp.int32),
            pltpu.VMEM((E_PER_W,), jnp.int32),
            pltpu.VMEM((E_PER_W,), jnp.float32),
            pltpu.VMEM((C1, TBL_W), jnp.bfloat16),
            pltpu.VMEM((C1, TBL_W), jnp.bfloat16),
            pltpu.VMEM((C1, TBL_W), jnp.bfloat16),
            pltpu.VMEM((C1, TBL_W), jnp.bfloat16),
            pltpu.VMEM((C1, 17), jnp.float32),
            pltpu.VMEM((16,), jnp.float32),
            pltpu.SemaphoreType.DMA,
            pltpu.SemaphoreType.DMA,
            pltpu.SemaphoreType.DMA,
            pltpu.SemaphoreType.DMA,
        ],
        compiler_params=pltpu.CompilerParams(use_tc_tiling_on_sc=False,
                                             needs_layout_passes=False),
    )
    return f(tn, src, dst, beta16)


S2 = 2000
NST = E_PER_T // S2
NCH2 = S2 // C2


def _sc_pass2_body(hn2_hbm, src_hbm, dst_hbm, ee_hbm,
                   acc_hbm, den_hbm,
                   sidx_st, didx_st, ee_st,
                   rowsA, rowsB, didx_v, denrows_v, zrows_v,
                   acc_sp, den_sp, semA, semB):
    cid = lax.axis_index("c")
    tid = lax.axis_index("s")
    iota = lax.iota(jnp.int32, 16)
    zero16 = jnp.zeros((16,), jnp.float32)

    def zb(r, _):
        for k in range(8):
            zrows_v[r, pl.ds(k * 16, 16)] = zero16
        return 0
    lax.fori_loop(0, 25, zb, 0)

    def zd(r, _):
        denrows_v[r, pl.ds(0, 16)] = zero16
        return 0
    lax.fori_loop(0, C2, zd, 0)

    r0 = tid * ROWS_PER_T
    for j in range(25):
        pltpu.sync_copy(zrows_v, acc_sp.at[pl.ds(r0 + j * 25, 25)])

    for j in range(25):
        pltpu.sync_copy(zrows_v.at[pl.ds(0, 25), pl.ds(0, 16)],
                        den_sp.at[pl.ds(r0 + j * 25, 25)])

    plsc.subcore_barrier()

    ebase = tid * E_PER_T
    coff = cid * N

    def start(c, rows, sem):
        pltpu.async_copy(hn2_hbm.at[sidx_st.at[pl.ds(c * C2, C2)]], rows, sem)

    def wait(rows, sem):
        pltpu.make_async_copy(hn2_hbm.at[pl.ds(0, C2)], rows, sem).wait()

    def compute(c, rows, do_den):
        o = c * C2
        for j in range(C2 // 16):
            didx_v[pl.ds(j * 16, 16)] = didx_st[pl.ds(o + j * 16, 16)]

        def scale_body(e, _):
            w = plsc.load_gather(ee_st, [jnp.full((16,), o + e, jnp.int32)])
            for k in range(8):
                rows[e, pl.ds(k * 16, 16)] = rows[e, pl.ds(k * 16, 16)] * w
            return 0
        lax.fori_loop(0, C2, scale_body, 0, unroll=4)

        pltpu.sync_copy(rows, acc_sp.at[didx_v], add=True)

        @pl.when(do_den)
        def _():
            for g in range(C2 // 16):
                e16 = ee_st[pl.ds(o + g * 16, 16)]
                plsc.store_scatter(
                    denrows_v, [g * 16 + iota, jnp.zeros((16,), jnp.int32)],
                    e16)
            pltpu.sync_copy(denrows_v, den_sp.at[didx_v], add=True)

    def stage_body(st, _):
        do_den = lax.rem(st, 2) == cid
        sb = ebase + st * S2
        pltpu.sync_copy(src_hbm.at[pl.ds(sb, S2)], sidx_st)
        pltpu.sync_copy(dst_hbm.at[pl.ds(sb, S2)], didx_st)
        pltpu.sync_copy(ee_hbm.at[pl.ds(sb, S2)], ee_st)

        def shift_body(j, _):
            v = sidx_st[pl.ds(j * 16, 16)]
            sidx_st[pl.ds(j * 16, 16)] = v + coff
            return 0
        lax.fori_loop(0, S2 // 16, shift_body, 0, unroll=4)

        start(0, rowsA, semA)

        def pair_body(it, _):
            cA = 2 * it
            wait(rowsA, semA)

            @pl.when(it < NCH2 // 2)
            def _():
                start(cA + 1, rowsB, semB)

            compute(cA, rowsA, do_den)

            @pl.when(it < NCH2 // 2)
            def _():
                wait(rowsB, semB)
                start(cA + 2, rowsA, semA)
                compute(cA + 1, rowsB, do_den)

            return 0

        lax.fori_loop(0, NCH2 // 2 + 1, pair_body, 0)
        return 0

    lax.fori_loop(0, NST, stage_body, 0)
    plsc.subcore_barrier()

    pltpu.sync_copy(acc_sp.at[pl.ds(r0, ROWS_PER_T)],
                    acc_hbm.at[cid, pl.ds(r0, ROWS_PER_T)])

    pltpu.sync_copy(den_sp.at[pl.ds(r0, ROWS_PER_T)],
                    den_hbm.at[cid, pl.ds(r0, ROWS_PER_T)])


def _sc_pass2(hn2, src, dst, ee):
    mesh = plsc.VectorSubcoreMesh(core_axis_name="c", subcore_axis_name="s")
    f = pl.kernel(
        _sc_pass2_body,
        out_type=[
            jax.ShapeDtypeStruct((NC, N, 128), jnp.float32),
            jax.ShapeDtypeStruct((NC, N, 16), jnp.float32),
        ],
        mesh=mesh,
        scratch_types=[
            pltpu.VMEM((S2,), jnp.int32),
            pltpu.VMEM((S2,), jnp.int32),
            pltpu.VMEM((S2,), jnp.float32),
            pltpu.VMEM((C2, 128), jnp.float32),
            pltpu.VMEM((C2, 128), jnp.float32),
            pltpu.VMEM((C2,), jnp.int32),
            pltpu.VMEM((C2, 16), jnp.float32),
            pltpu.VMEM((25, 128), jnp.float32),
            pltpu.VMEM_SHARED((N, 128), jnp.float32),
            pltpu.VMEM_SHARED((N, 16), jnp.float32),
            pltpu.SemaphoreType.DMA,
            pltpu.SemaphoreType.DMA,
        ],
        compiler_params=pltpu.CompilerParams(use_tc_tiling_on_sc=False,
                                             needs_layout_passes=False),
    )
    return f(hn2, src, dst, ee)


def _tc_final_body(alo_ref, ahi_ref, den_ref, w2_ref, b2_ref, out_ref):
    den = den_ref[0, :, 0:1] + den_ref[1, :, 0:1]
    inv = 1.0 / jnp.maximum(den, 1e-16)
    olo = alo_ref[0] * inv
    ohi = ahi_ref[0] * inv
    w2 = w2_ref[...]
    logits = (lax.dot_general(olo, w2[:, :128], (((1,), (1,)), ((), ())),
                              precision=lax.Precision.HIGHEST)
              + lax.dot_general(ohi, w2[:, 128:], (((1,), (1,)), ((), ())),
                                precision=lax.Precision.HIGHEST))
    logits = logits + b2_ref[...][None, :]
    m = jnp.max(logits, axis=1, keepdims=True)
    lse = jnp.log(jnp.sum(jnp.exp(logits - m), axis=1, keepdims=True)) + m
    out_ref[...] = logits - lse


def _tc_final(acc2, den, W2, b2):
    blk = 1000
    grid = N // blk
    return pl.pallas_call(
        _tc_final_body,
        grid=(grid,),
        in_specs=[
            pl.BlockSpec((1, blk, 128), lambda i: (0, i, 0)),
            pl.BlockSpec((1, blk, 128), lambda i: (1, i, 0)),
            pl.BlockSpec((2, blk, 16), lambda i: (0, i, 0)),
            pl.BlockSpec((N_CLS, D_HID), lambda i: (0, 0)),
            pl.BlockSpec((N_CLS,), lambda i: (0,)),
        ],
        out_specs=pl.BlockSpec((blk, N_CLS), lambda i: (i, 0)),
        out_shape=jax.ShapeDtypeStruct((N, N_CLS), jnp.float32),
    )(acc2, acc2, den, W2, b2)


def kernel(x, edge_index, edge_weight, W1, b1, beta, W2, b2):
    del edge_weight
    ei = jnp.asarray(edge_index, jnp.int32)
    src = ei[0]
    dst = ei[1]
    beta16 = jnp.broadcast_to(beta.astype(jnp.float32), (16,))

    tn, lohi = _tc_prep(x, W1, b1)
    hn2 = lohi.reshape(2 * N, 128)

    (ee,) = _sc_pass1(tn, src, dst, beta16)
    acc2, den = _sc_pass2(hn2, src, dst, ee)

    return _tc_final(acc2, den, W2, b2)
